# baseline (speedup 1.0000x reference)
import jax
import jax.numpy as jnp
from jax import lax
from jax.experimental import pallas as pl
from jax.experimental.pallas import tpu as pltpu
from jax.experimental.pallas import tpu_sc as plsc

N = 10000
E = 320000
D = 128
NC = 2
NS = 16
NW = NC * NS
NP = 10240
RPT = NP // NS
EP = E // NW
C = 80
NCH = EP // C
RA = 640
RL = N - (NS - 1) * RA
NB = 25
RB = N // NB

_sc_mesh = plsc.VectorSubcoreMesh(
    core_axis_name="c", subcore_axis_name="s", num_cores=NC, num_subcores=NS
)


def _deg_body(col2_hbm, deg0_hbm, deg1_hbm, acc, idx_a, ones_v, zero_v):
    c = lax.axis_index("c")
    s = lax.axis_index("s")
    wid = c * NS + s

    @pl.loop(0, RPT // 16)
    def _zfill(i):
        zero_v[pl.ds(i * 16, 16)] = jnp.zeros((16,), jnp.float32)

    @pl.loop(0, C // 16)
    def _ofill(i):
        ones_v[pl.ds(i * 16, 16)] = jnp.ones((16,), jnp.float32)

    pltpu.sync_copy(zero_v, acc.at[pl.ds(s * RPT, RPT)])
    pltpu.sync_copy(col2_hbm.at[wid], idx_a)
    plsc.subcore_barrier()

    @pl.loop(0, NCH)
    def _chunk(k):
        pltpu.sync_copy(ones_v, acc.at[idx_a.at[k]], add=True)

    plsc.subcore_barrier()

    @pl.when(c == 0)
    def _dump0():
        pltpu.sync_copy(acc.at[pl.ds(s * RPT, RPT)], deg0_hbm.at[pl.ds(s * RPT, RPT)])

    @pl.when(c == 1)
    def _dump1():
        pltpu.sync_copy(acc.at[pl.ds(s * RPT, RPT)], deg1_hbm.at[pl.ds(s * RPT, RPT)])


_deg_kernel = pl.kernel(
    _deg_body,
    out_type=[
        jax.ShapeDtypeStruct((NP,), jnp.float32),
        jax.ShapeDtypeStruct((NP,), jnp.float32),
    ],
    mesh=_sc_mesh,
    scratch_types=[
        pltpu.VMEM_SHARED((NP,), jnp.float32),
        pltpu.VMEM((NCH, C), jnp.int32),
        pltpu.VMEM((C,), jnp.float32),
        pltpu.VMEM((RPT,), jnp.float32),
    ],
)


def _spmm_body(hs_hbm, pk_hbm, z_hbm, p0_hbm, p1_hbm,
               acc, k0b, k1b, k2b, k3b, rb0, rb1, rb2, rb3,
               cb0, cb1, cb2, cb3, r0, r1, r2, r3,
               i0, i1, i2, i3, g0, g1, g2, g3, s0, s1, s2, s3):
    c = lax.axis_index("c")
    s = lax.axis_index("s")
    wid = c * NS + s
    ebase = wid * EP

    pkb = (k0b, k1b, k2b, k3b)
    rbs = (rb0, rb1, rb2, rb3)
    cbs = (cb0, cb1, cb2, cb3)
    rows = (r0, r1, r2, r3)
    isems = (i0, i1, i2, i3)
    gsems = (g0, g1, g2, g3)
    ssems = (s0, s1, s2, s3)

    @pl.when(s < NS - 1)
    def _zmain():
        pltpu.sync_copy(z_hbm, acc.at[pl.ds(s * RA, RA)])

    @pl.when(s == NS - 1)
    def _ztail():
        pltpu.sync_copy(z_hbm.at[pl.ds(0, RL)], acc.at[pl.ds(s * RA, RL)])

    plsc.subcore_barrier()

    def _unpack(k, j):
        for i in range(C // 16):
            v = pkb[j][pl.ds(i * 16, 16)]
            rbs[j][pl.ds(i * 16, 16)] = v & 0xFFFF
            cbs[j][pl.ds(i * 16, 16)] = lax.shift_right_logical(v, 16)

    for j in (0, 1, 2, 3):
        pltpu.async_copy(pk_hbm.at[pl.ds(ebase + j * C, C)], pkb[j], isems[j])
    for j in (0, 1):
        pltpu.make_async_copy(pk_hbm.at[pl.ds(ebase + j * C, C)], pkb[j], isems[j]).wait()
        _unpack(j, j)
        pltpu.async_copy(hs_hbm.at[rbs[j]], rows[j], gsems[j])

    @pl.loop(0, NCH - 1, step=4)
    def _chunk(k0):
        for u in (0, 1, 2, 3):
            k = k0 + u
            j = u % 4
            j2 = (u + 2) % 4
            pltpu.make_async_copy(hs_hbm.at[rbs[j]], rows[j], gsems[j]).wait()
            pltpu.async_copy(rows[j], acc.at[cbs[j]], ssems[j], add=True)

            @pl.when(k >= 2)
            def _drain():
                pltpu.make_async_copy(rows[j2], acc.at[cbs[j2]], ssems[j2]).wait()

            @pl.when(k + 2 < NCH)
            def _next():
                pltpu.make_async_copy(
                    pk_hbm.at[pl.ds(ebase + (k + 2) * C, C)], pkb[j2], isems[j2]
                ).wait()
                _unpack(k + 2, j2)
                pltpu.async_copy(hs_hbm.at[rbs[j2]], rows[j2], gsems[j2])

            @pl.when(k + 4 < NCH)
            def _refill():
                pltpu.async_copy(pk_hbm.at[pl.ds(ebase + (k + 4) * C, C)], pkb[j], isems[j])

    kl = NCH - 1
    jl = kl % 4
    pltpu.make_async_copy(hs_hbm.at[rbs[jl]], rows[jl], gsems[jl]).wait()
    pltpu.async_copy(rows[jl], acc.at[cbs[jl]], ssems[jl], add=True)
    for k in (NCH - 3, NCH - 2, NCH - 1):
        j = k % 4
        pltpu.make_async_copy(rows[j], acc.at[cbs[j]], ssems[j]).wait()

    plsc.subcore_barrier()

    def _dump(pout):
        @pl.when(s < NS - 1)
        def _dmain():
            pltpu.sync_copy(acc.at[pl.ds(s * RA, RA)], pout.at[pl.ds(s * RA, RA)])

        @pl.when(s == NS - 1)
        def _dtail():
            pltpu.sync_copy(acc.at[pl.ds(s * RA, RL)], pout.at[pl.ds(s * RA, RL)])

    @pl.when(c == 0)
    def _dump0():
        _dump(p0_hbm)

    @pl.when(c == 1)
    def _dump1():
        _dump(p1_hbm)


_spmm_kernel = pl.kernel(
    _spmm_body,
    out_type=[
        jax.ShapeDtypeStruct((N, D), jnp.float32),
        jax.ShapeDtypeStruct((N, D), jnp.float32),
    ],
    mesh=_sc_mesh,
    scratch_types=(
        [pltpu.VMEM_SHARED((N, D), jnp.float32)]
        + [pltpu.VMEM((C,), jnp.int32) for _ in range(4)]
        + [pltpu.VMEM((C,), jnp.int32) for _ in range(8)]
        + [pltpu.VMEM((C, D), jnp.float32) for _ in range(4)]
        + [pltpu.SemaphoreType.DMA for _ in range(12)]
    ),
)


NCHF = (E // NS) // C


def _front_body(x_hbm, pk_hbm, z_hbm, hs00_hbm, hs01_hbm, dinv_hbm,
                p0_hbm, p1_hbm,
                acc, dacc, k0b, k1b, k2b, k3b, rb0, rb1, rb2, rb3,
                cb0, cb1, cb2, cb3, r0, r1, dv, ones_v,
                i0, i1, i2, i3, g0, g1):
    c = lax.axis_index("c")
    s = lax.axis_index("s")
    wid = c * NS + s

    pkb = (k0b, k1b, k2b, k3b)
    rbs = (rb0, rb1, rb2, rb3)
    cbs = (cb0, cb1, cb2, cb3)
    rows = (r0, r1)
    isems = (i0, i1, i2, i3)
    gsems = (g0, g1)

    @pl.loop(0, RA // 16)
    def _zfill(i):
        dv[pl.ds(i * 16, 16)] = jnp.zeros((16,), jnp.float32)

    @pl.loop(0, C // 16)
    def _ofill(i):
        ones_v[pl.ds(i * 16, 16)] = jnp.ones((16,), jnp.float32)

    @pl.when(s < NS - 1)
    def _zmain():
        pltpu.sync_copy(z_hbm, acc.at[pl.ds(s * RA, RA)])
        pltpu.sync_copy(dv, dacc.at[pl.ds(s * RA, RA)])

    @pl.when(s == NS - 1)
    def _ztail():
        pltpu.sync_copy(z_hbm.at[pl.ds(0, RL)], acc.at[pl.ds(s * RA, RL)])
        pltpu.sync_copy(dv.at[pl.ds(0, RL)], dacc.at[pl.ds(s * RA, RL)])

    plsc.subcore_barrier()

    dbase = s * (E // NS)
    for j in (0, 1, 2, 3):
        pltpu.async_copy(pk_hbm.at[pl.ds(dbase + j * C, C)], pkb[j], isems[j])

    @pl.loop(0, NCHF - 2, step=4)
    def _dchunk(k0):
        for u in (0, 1, 2, 3):
            k = k0 + u
            j = u % 4
            pltpu.make_async_copy(
                pk_hbm.at[pl.ds(dbase + k * C, C)], pkb[j], isems[j]).wait()
            for i in range(C // 16):
                cbs[j][pl.ds(i * 16, 16)] = lax.shift_right_logical(
                    pkb[j][pl.ds(i * 16, 16)], 16)
            pltpu.sync_copy(ones_v, dacc.at[cbs[j]], add=True)

            @pl.when(k + 4 < NCHF)
            def _refill():
                pltpu.async_copy(
                    pk_hbm.at[pl.ds(dbase + (k + 4) * C, C)], pkb[j], isems[j])

    for k in (NCHF - 2, NCHF - 1):
        j = k % 4
        pltpu.make_async_copy(
            pk_hbm.at[pl.ds(dbase + k * C, C)], pkb[j], isems[j]).wait()
        for i in range(C // 16):
            cbs[j][pl.ds(i * 16, 16)] = lax.shift_right_logical(
                pkb[j][pl.ds(i * 16, 16)], 16)
        pltpu.sync_copy(ones_v, dacc.at[cbs[j]], add=True)

    plsc.subcore_barrier()

    @pl.when(s < NS - 1)
    def _vmain():
        pltpu.sync_copy(dacc.at[pl.ds(s * RA, RA)], dv)

    @pl.when(s == NS - 1)
    def _vtail():
        pltpu.sync_copy(dacc.at[pl.ds(s * RA, RL)], dv.at[pl.ds(0, RL)])

    @pl.loop(0, RA // 16)
    def _newton(i):
        v = dv[pl.ds(i * 16, 16)] + 1.0
        sq = 0.5 * (v + 1.0)
        for _ in range(14):
            sq = 0.5 * (sq + v / sq)
        dv[pl.ds(i * 16, 16)] = 1.0 / sq

    @pl.when(jnp.logical_and(c == 0, s < NS - 1))
    def _dvmain():
        pltpu.sync_copy(dv, dinv_hbm.at[pl.ds(s * RA, RA)])

    @pl.when(jnp.logical_and(c == 0, s == NS - 1))
    def _dvtail():
        pltpu.sync_copy(dv.at[pl.ds(0, RL)], dinv_hbm.at[pl.ds(s * RA, RL)])

    def _scale_out(hs_hbm):
        def _blk(bi):
            base = s * RA + bi * C
            pltpu.sync_copy(x_hbm.at[pl.ds(base, C)], rows[0])

            @pl.loop(0, C // 16)
            def _grp(g):
                dvec = dv[pl.ds(bi * C + g * 16, 16)]
                for rr in range(16):
                    d = lax.gather(
                        dvec, jnp.full((16, 1), rr, jnp.int32),
                        dimension_numbers=lax.GatherDimensionNumbers(
                            offset_dims=(), collapsed_slice_dims=(0,),
                            start_index_map=(0,)),
                        slice_sizes=(1,),
                        mode=lax.GatherScatterMode.PROMISE_IN_BOUNDS)
                    for jj in range(D // 16):
                        rows[0][g * 16 + rr, pl.ds(jj * 16, 16)] = (
                            rows[0][g * 16 + rr, pl.ds(jj * 16, 16)] * d)

            pltpu.sync_copy(rows[0], hs_hbm.at[pl.ds(base, C)])

        @pl.when(s < NS - 1)
        def _hmain():
            pl.loop(0, RA // C)(_blk)

        @pl.when(s == NS - 1)
        def _htail():
            pl.loop(0, RL // C)(_blk)

    def _spmm1(hs_hbm, pout):
        sbase = wid * EP
        for j in (0, 1, 2, 3):
            pltpu.async_copy(pk_hbm.at[pl.ds(sbase + j * C, C)], pkb[j], isems[j])
        for j in (0, 1):
            pltpu.make_async_copy(
                pk_hbm.at[pl.ds(sbase + j * C, C)], pkb[j], isems[j]).wait()
            for i in range(C // 16):
                v = pkb[j][pl.ds(i * 16, 16)]
                rbs[j][pl.ds(i * 16, 16)] = v & 0xFFFF
                cbs[j][pl.ds(i * 16, 16)] = lax.shift_right_logical(v, 16)
            pltpu.async_copy(hs_hbm.at[rbs[j]], rows[j], gsems[j])

        @pl.loop(0, NCH - 1, step=4)
        def _chunk(k0):
            for u in (0, 1, 2, 3):
                k = k0 + u
                j = u % 4
                j2 = (u + 2) % 4
                b = u % 2
                pltpu.make_async_copy(hs_hbm.at[rbs[j]], rows[b], gsems[b]).wait()
                pltpu.sync_copy(rows[b], acc.at[cbs[j]], add=True)

                @pl.when(k + 4 < NCH)
                def _refill():
                    pltpu.async_copy(
                        pk_hbm.at[pl.ds(sbase + (k + 4) * C, C)], pkb[j], isems[j])

                @pl.when(k + 2 < NCH)
                def _next():
                    pltpu.make_async_copy(
                        pk_hbm.at[pl.ds(sbase + (k + 2) * C, C)],
                        pkb[j2], isems[j2]).wait()
                    for i in range(C // 16):
                        v = pkb[j2][pl.ds(i * 16, 16)]
                        rbs[j2][pl.ds(i * 16, 16)] = v & 0xFFFF
                        cbs[j2][pl.ds(i * 16, 16)] = lax.shift_right_logical(v, 16)
                    pltpu.async_copy(hs_hbm.at[rbs[j2]], rows[b], gsems[b])

        kl = NCH - 1
        jl = kl % 4
        bl = kl % 2
        pltpu.make_async_copy(hs_hbm.at[rbs[jl]], rows[bl], gsems[bl]).wait()
        pltpu.sync_copy(rows[bl], acc.at[cbs[jl]], add=True)

        plsc.subcore_barrier()

        @pl.when(s < NS - 1)
        def _dmain():
            pltpu.sync_copy(acc.at[pl.ds(s * RA, RA)], pout.at[pl.ds(s * RA, RA)])

        @pl.when(s == NS - 1)
        def _dtail():
            pltpu.sync_copy(acc.at[pl.ds(s * RA, RL)], pout.at[pl.ds(s * RA, RL)])

    @pl.when(c == 0)
    def _core0():
        _scale_out(hs00_hbm)
        plsc.subcore_barrier()
        _spmm1(hs00_hbm, p0_hbm)

    @pl.when(c == 1)
    def _core1():
        _scale_out(hs01_hbm)
        plsc.subcore_barrier()
        _spmm1(hs01_hbm, p1_hbm)


_front_kernel = pl.kernel(
    _front_body,
    out_type=[
        jax.ShapeDtypeStruct((N, D), jnp.float32),
        jax.ShapeDtypeStruct((N, D), jnp.float32),
        jax.ShapeDtypeStruct((N,), jnp.float32),
        jax.ShapeDtypeStruct((N, D), jnp.float32),
        jax.ShapeDtypeStruct((N, D), jnp.float32),
    ],
    mesh=_sc_mesh,
    scratch_types=(
        [pltpu.VMEM_SHARED((N, D), jnp.float32),
         pltpu.VMEM_SHARED((N,), jnp.float32)]
        + [pltpu.VMEM((C,), jnp.int32) for _ in range(4)]
        + [pltpu.VMEM((C,), jnp.int32) for _ in range(8)]
        + [pltpu.VMEM((C, D), jnp.float32) for _ in range(2)]
        + [pltpu.VMEM((RA,), jnp.float32),
           pltpu.VMEM((C,), jnp.float32)]
        + [pltpu.SemaphoreType.DMA for _ in range(6)]
    ),
)


def _prep_body(d0_ref, d1_ref, x_ref, dinv_ref, hs0_ref):
    deg = d0_ref[...] + d1_ref[...] + 1.0
    dinv = lax.rsqrt(deg)
    dinv_ref[...] = dinv
    hs0_ref[...] = dinv * x_ref[...]


_prep_kernel = pl.pallas_call(
    _prep_body,
    grid=(NB,),
    in_specs=[
        pl.BlockSpec((RB, 1), lambda i: (i, 0)),
        pl.BlockSpec((RB, 1), lambda i: (i, 0)),
        pl.BlockSpec((RB, D), lambda i: (i, 0)),
    ],
    out_specs=[
        pl.BlockSpec((RB, 1), lambda i: (i, 0)),
        pl.BlockSpec((RB, D), lambda i: (i, 0)),
    ],
    out_shape=[
        jax.ShapeDtypeStruct((N, 1), jnp.float32),
        jax.ShapeDtypeStruct((N, D), jnp.float32),
    ],
)


def _mid_body(dinv_ref, p0_ref, p1_ref, hs0_ref, h1_ref, hs1_ref):
    agg = p0_ref[...] + p1_ref[...] + hs0_ref[...]
    dinv = dinv_ref[...]
    h1 = dinv * agg
    h1_ref[...] = h1
    hs1_ref[...] = dinv * h1


_mid_kernel = pl.pallas_call(
    _mid_body,
    grid=(NB,),
    in_specs=[
        pl.BlockSpec((RB, 1), lambda i: (i, 0)),
        pl.BlockSpec((RB, D), lambda i: (i, 0)),
        pl.BlockSpec((RB, D), lambda i: (i, 0)),
        pl.BlockSpec((RB, D), lambda i: (i, 0)),
    ],
    out_specs=[
        pl.BlockSpec((RB, D), lambda i: (i, 0)),
        pl.BlockSpec((RB, D), lambda i: (i, 0)),
    ],
    out_shape=[
        jax.ShapeDtypeStruct((N, D), jnp.float32),
        jax.ShapeDtypeStruct((N, D), jnp.float32),
    ],
)


def _out_body(x_ref, h1_ref, q0_ref, q1_ref, hs1_ref, dinv_ref,
              w0_ref, w1_ref, w2_ref, b_ref, o_ref):
    h2 = dinv_ref[...] * (q0_ref[...] + q1_ref[...] + hs1_ref[...])
    acc = jnp.dot(x_ref[...], w0_ref[...], preferred_element_type=jnp.float32)
    acc = acc + jnp.dot(h1_ref[...], w1_ref[...], preferred_element_type=jnp.float32)
    acc = acc + jnp.dot(h2, w2_ref[...], preferred_element_type=jnp.float32)
    o_ref[...] = acc + b_ref[...]


_out_kernel = pl.pallas_call(
    _out_body,
    grid=(NB,),
    in_specs=[
        pl.BlockSpec((RB, D), lambda i: (i, 0)),
        pl.BlockSpec((RB, D), lambda i: (i, 0)),
        pl.BlockSpec((RB, D), lambda i: (i, 0)),
        pl.BlockSpec((RB, D), lambda i: (i, 0)),
        pl.BlockSpec((RB, D), lambda i: (i, 0)),
        pl.BlockSpec((RB, 1), lambda i: (i, 0)),
        pl.BlockSpec((D, D), lambda i: (0, 0)),
        pl.BlockSpec((D, D), lambda i: (0, 0)),
        pl.BlockSpec((D, D), lambda i: (0, 0)),
        pl.BlockSpec((1, D), lambda i: (0, 0)),
    ],
    out_specs=pl.BlockSpec((RB, D), lambda i: (i, 0)),
    out_shape=jax.ShapeDtypeStruct((N, D), jnp.float32),
)


@jax.jit
def kernel(x, edge_index, W, b):
    packed = edge_index[0] | (edge_index[1] << 16)
    zrows = jnp.zeros((RA, D), jnp.float32)

    hs00, hs01, dinv, p0, p1 = _front_kernel(x, packed, zrows)
    dinv = dinv.reshape(N, 1)
    h1, hs1 = _mid_kernel(dinv, p0, p1, hs00)
    q0, q1 = _spmm_kernel(hs1, packed, zrows)
    Wt = W.T
    return _out_kernel(x, h1, q0, q1, hs1, dinv,
                       Wt[:D], Wt[D:2 * D], Wt[2 * D:], b.reshape(1, D))


# speedup vs baseline: 1.0731x; 1.0731x over previous
import functools

import jax
import jax.numpy as jnp
from jax import lax
from jax.experimental import pallas as pl
from jax.experimental.pallas import tpu as pltpu
from jax.experimental.pallas import tpu_sc as plsc

N = 10000
E = 320000
D = 128
NC = 2
NS = 16
NW = NC * NS
NP = 10240
RPT = NP // NS
EP = E // NW
C = 80
NCH = EP // C
NB = 16
RB = NP // NB

_sc_mesh = plsc.VectorSubcoreMesh(
    core_axis_name="c", subcore_axis_name="s", num_cores=NC, num_subcores=NS
)


def _deg_body(col2_hbm, deg0_hbm, deg1_hbm, acc, idx_a, ones_v, zero_v):
    c = lax.axis_index("c")
    s = lax.axis_index("s")
    wid = c * NS + s

    @pl.loop(0, RPT // 16)
    def _zfill(i):
        zero_v[pl.ds(i * 16, 16)] = jnp.zeros((16,), jnp.float32)

    @pl.loop(0, C // 16)
    def _ofill(i):
        ones_v[pl.ds(i * 16, 16)] = jnp.ones((16,), jnp.float32)

    pltpu.sync_copy(zero_v, acc.at[pl.ds(s * RPT, RPT)])
    pltpu.sync_copy(col2_hbm.at[wid], idx_a)
    plsc.subcore_barrier()

    @pl.loop(0, NCH)
    def _chunk(k):
        pltpu.sync_copy(ones_v, acc.at[idx_a.at[k]], add=True)

    plsc.subcore_barrier()

    @pl.when(c == 0)
    def _dump0():
        pltpu.sync_copy(acc.at[pl.ds(s * RPT, RPT)], deg0_hbm.at[pl.ds(s * RPT, RPT)])

    @pl.when(c == 1)
    def _dump1():
        pltpu.sync_copy(acc.at[pl.ds(s * RPT, RPT)], deg1_hbm.at[pl.ds(s * RPT, RPT)])


_deg_kernel = pl.kernel(
    _deg_body,
    out_type=[
        jax.ShapeDtypeStruct((NP,), jnp.float32),
        jax.ShapeDtypeStruct((NP,), jnp.float32),
    ],
    mesh=_sc_mesh,
    scratch_types=[
        pltpu.VMEM_SHARED((NP,), jnp.float32),
        pltpu.VMEM((NCH, C), jnp.int32),
        pltpu.VMEM((C,), jnp.float32),
        pltpu.VMEM((RPT,), jnp.float32),
    ],
)


def _spmm_body(hs_hbm, eidx_hbm, z_hbm, p0_hbm, p1_hbm,
               acc, i0, i1, i2, i3, rows0, rows1,
               si0, si1, si2, si3, sg0, sg1):
    c = lax.axis_index("c")
    s = lax.axis_index("s")
    wid = c * NS + s

    idx = (i0, i1, i2, i3)
    isems = (si0, si1, si2, si3)
    rows = (rows0, rows1)
    gsems = (sg0, sg1)

    pltpu.sync_copy(z_hbm, acc.at[pl.ds(s * RPT, RPT)])
    plsc.subcore_barrier()

    for j in (0, 1, 2, 3):
        pltpu.async_copy(eidx_hbm.at[wid, j], idx[j], isems[j])
    for b in (0, 1):
        pltpu.make_async_copy(eidx_hbm.at[wid, b], idx[b], isems[b]).wait()
        pltpu.async_copy(hs_hbm.at[idx[b].at[0]], rows[b], gsems[b])

    @pl.loop(0, NCH - 1, step=4)
    def _chunk(k0):
        for u in (0, 1, 2, 3):
            k = k0 + u
            b = u % 2
            j = u % 4
            j2 = (u + 2) % 4
            pltpu.make_async_copy(hs_hbm.at[idx[j].at[0]], rows[b], gsems[b]).wait()
            pltpu.sync_copy(rows[b], acc.at[idx[j].at[1]], add=True)

            @pl.when(k + 4 < NCH)
            def _refill():
                pltpu.async_copy(eidx_hbm.at[wid, k + 4], idx[j], isems[j])

            @pl.when(k + 2 < NCH)
            def _launch():
                pltpu.make_async_copy(eidx_hbm.at[wid, k + 2], idx[j2], isems[j2]).wait()
                pltpu.async_copy(hs_hbm.at[idx[j2].at[0]], rows[b], gsems[b])

    kl = NCH - 1
    bl = kl % 2
    jl = kl % 4
    pltpu.make_async_copy(hs_hbm.at[idx[jl].at[0]], rows[bl], gsems[bl]).wait()
    pltpu.sync_copy(rows[bl], acc.at[idx[jl].at[1]], add=True)

    plsc.subcore_barrier()

    @pl.when(c == 0)
    def _dump0():
        pltpu.sync_copy(acc.at[pl.ds(s * RPT, RPT)], p0_hbm.at[pl.ds(s * RPT, RPT)])

    @pl.when(c == 1)
    def _dump1():
        pltpu.sync_copy(acc.at[pl.ds(s * RPT, RPT)], p1_hbm.at[pl.ds(s * RPT, RPT)])


_spmm_kernel = pl.kernel(
    _spmm_body,
    out_type=[
        jax.ShapeDtypeStruct((NP, D), jnp.float32),
        jax.ShapeDtypeStruct((NP, D), jnp.float32),
    ],
    mesh=_sc_mesh,
    scratch_types=[
        pltpu.VMEM_SHARED((NP, D), jnp.float32),
        pltpu.VMEM((2, C), jnp.int32),
        pltpu.VMEM((2, C), jnp.int32),
        pltpu.VMEM((2, C), jnp.int32),
        pltpu.VMEM((2, C), jnp.int32),
        pltpu.VMEM((C, D), jnp.float32),
        pltpu.VMEM((C, D), jnp.float32),
        pltpu.SemaphoreType.DMA,
        pltpu.SemaphoreType.DMA,
        pltpu.SemaphoreType.DMA,
        pltpu.SemaphoreType.DMA,
        pltpu.SemaphoreType.DMA,
        pltpu.SemaphoreType.DMA,
    ],
)


def _prep_body(d0_ref, d1_ref, x_ref, dinv_ref, hs0_ref):
    deg = d0_ref[...] + d1_ref[...] + 1.0
    dinv = lax.rsqrt(deg)
    dinv_ref[...] = dinv
    hs0_ref[...] = dinv * x_ref[...]


_prep_kernel = pl.pallas_call(
    _prep_body,
    grid=(NB,),
    in_specs=[
        pl.BlockSpec((RB, 1), lambda i: (i, 0)),
        pl.BlockSpec((RB, 1), lambda i: (i, 0)),
        pl.BlockSpec((RB, D), lambda i: (i, 0)),
    ],
    out_specs=[
        pl.BlockSpec((RB, 1), lambda i: (i, 0)),
        pl.BlockSpec((RB, D), lambda i: (i, 0)),
    ],
    out_shape=[
        jax.ShapeDtypeStruct((NP, 1), jnp.float32),
        jax.ShapeDtypeStruct((NP, D), jnp.float32),
    ],
)


def _mid_body(dinv_ref, p0_ref, p1_ref, hs0_ref, h1_ref, hs1_ref):
    agg = p0_ref[...] + p1_ref[...] + hs0_ref[...]
    dinv = dinv_ref[...]
    h1 = dinv * agg
    h1_ref[...] = h1
    hs1_ref[...] = dinv * h1


_mid_kernel = pl.pallas_call(
    _mid_body,
    grid=(NB,),
    in_specs=[
        pl.BlockSpec((RB, 1), lambda i: (i, 0)),
        pl.BlockSpec((RB, D), lambda i: (i, 0)),
        pl.BlockSpec((RB, D), lambda i: (i, 0)),
        pl.BlockSpec((RB, D), lambda i: (i, 0)),
    ],
    out_specs=[
        pl.BlockSpec((RB, D), lambda i: (i, 0)),
        pl.BlockSpec((RB, D), lambda i: (i, 0)),
    ],
    out_shape=[
        jax.ShapeDtypeStruct((NP, D), jnp.float32),
        jax.ShapeDtypeStruct((NP, D), jnp.float32),
    ],
)


def _out_body(x_ref, h1_ref, q0_ref, q1_ref, hs1_ref, dinv_ref,
              w0_ref, w1_ref, w2_ref, b_ref, o_ref):
    h2 = dinv_ref[...] * (q0_ref[...] + q1_ref[...] + hs1_ref[...])
    acc = jnp.dot(x_ref[...], w0_ref[...], preferred_element_type=jnp.float32)
    acc = acc + jnp.dot(h1_ref[...], w1_ref[...], preferred_element_type=jnp.float32)
    acc = acc + jnp.dot(h2, w2_ref[...], preferred_element_type=jnp.float32)
    o_ref[...] = acc + b_ref[...]


_out_kernel = pl.pallas_call(
    _out_body,
    grid=(NB,),
    in_specs=[
        pl.BlockSpec((RB, D), lambda i: (i, 0)),
        pl.BlockSpec((RB, D), lambda i: (i, 0)),
        pl.BlockSpec((RB, D), lambda i: (i, 0)),
        pl.BlockSpec((RB, D), lambda i: (i, 0)),
        pl.BlockSpec((RB, D), lambda i: (i, 0)),
        pl.BlockSpec((RB, 1), lambda i: (i, 0)),
        pl.BlockSpec((D, D), lambda i: (0, 0)),
        pl.BlockSpec((D, D), lambda i: (0, 0)),
        pl.BlockSpec((D, D), lambda i: (0, 0)),
        pl.BlockSpec((1, D), lambda i: (0, 0)),
    ],
    out_specs=pl.BlockSpec((RB, D), lambda i: (i, 0)),
    out_shape=jax.ShapeDtypeStruct((NP, D), jnp.float32),
)


@jax.jit
def kernel(x, edge_index, W, b):
    col2 = edge_index[1].reshape(NW, NCH, C)
    eidx = edge_index.reshape(2, NW, NCH, C).transpose(1, 2, 0, 3)
    xp = jnp.pad(x, ((0, NP - N), (0, 0)))
    zrows = jnp.zeros((RPT, D), jnp.float32)

    d0, d1 = _deg_kernel(col2)
    dinv, hs0 = _prep_kernel(d0.reshape(NP, 1), d1.reshape(NP, 1), xp)
    p0, p1 = _spmm_kernel(hs0, eidx, zrows)
    h1, hs1 = _mid_kernel(dinv, p0, p1, hs0)
    q0, q1 = _spmm_kernel(hs1, eidx, zrows)
    Wt = W.T
    out = _out_kernel(xp, h1, q0, q1, hs1, dinv,
                      Wt[:D], Wt[D:2 * D], Wt[2 * D:], b.reshape(1, D))
    return out[:N]


# speedup vs baseline: 1.0807x; 1.0070x over previous
import functools

import jax
import jax.numpy as jnp
from jax import lax
from jax.experimental import pallas as pl
from jax.experimental.pallas import tpu as pltpu
from jax.experimental.pallas import tpu_sc as plsc

N = 10000
E = 320000
D = 128
NC = 2
NS = 16
NW = NC * NS
NP = 10240
RPT = NP // NS
EP = E // NW
C = 80
NCH = EP // C
NB = 16
RB = NP // NB

_sc_mesh = plsc.VectorSubcoreMesh(
    core_axis_name="c", subcore_axis_name="s", num_cores=NC, num_subcores=NS
)


def _deg_body(col2_hbm, deg0_hbm, deg1_hbm, acc, idx_a, ones_v, zero_v):
    c = lax.axis_index("c")
    s = lax.axis_index("s")
    wid = c * NS + s

    @pl.loop(0, RPT // 16)
    def _zfill(i):
        zero_v[pl.ds(i * 16, 16)] = jnp.zeros((16,), jnp.float32)

    @pl.loop(0, C // 16)
    def _ofill(i):
        ones_v[pl.ds(i * 16, 16)] = jnp.ones((16,), jnp.float32)

    pltpu.sync_copy(zero_v, acc.at[pl.ds(s * RPT, RPT)])
    pltpu.sync_copy(col2_hbm.at[wid], idx_a)
    plsc.subcore_barrier()

    @pl.loop(0, NCH)
    def _chunk(k):
        pltpu.sync_copy(ones_v, acc.at[idx_a.at[k]], add=True)

    plsc.subcore_barrier()

    @pl.when(c == 0)
    def _dump0():
        pltpu.sync_copy(acc.at[pl.ds(s * RPT, RPT)], deg0_hbm.at[pl.ds(s * RPT, RPT)])

    @pl.when(c == 1)
    def _dump1():
        pltpu.sync_copy(acc.at[pl.ds(s * RPT, RPT)], deg1_hbm.at[pl.ds(s * RPT, RPT)])


_deg_kernel = pl.kernel(
    _deg_body,
    out_type=[
        jax.ShapeDtypeStruct((NP,), jnp.float32),
        jax.ShapeDtypeStruct((NP,), jnp.float32),
    ],
    mesh=_sc_mesh,
    scratch_types=[
        pltpu.VMEM_SHARED((NP,), jnp.float32),
        pltpu.VMEM((NCH, C), jnp.int32),
        pltpu.VMEM((C,), jnp.float32),
        pltpu.VMEM((RPT,), jnp.float32),
    ],
)


def _spmm_body(hs_hbm, eidx_hbm, z_hbm, p0_hbm, p1_hbm,
               acc, i0, i1, i2, i3, rows0, rows1,
               si0, si1, si2, si3, sg0, sg1):
    c = lax.axis_index("c")
    s = lax.axis_index("s")
    wid = c * NS + s

    idx = (i0, i1, i2, i3)
    isems = (si0, si1, si2, si3)
    rows = (rows0, rows1)
    gsems = (sg0, sg1)

    for j in (0, 1, 2, 3):
        pltpu.async_copy(eidx_hbm.at[wid, j], idx[j], isems[j])
    pltpu.sync_copy(z_hbm, acc.at[pl.ds(s * RPT, RPT)])
    for b in (0, 1):
        pltpu.make_async_copy(eidx_hbm.at[wid, b], idx[b], isems[b]).wait()
        pltpu.async_copy(hs_hbm.at[idx[b].at[0]], rows[b], gsems[b])
    plsc.subcore_barrier()

    @pl.loop(0, NCH - 1, step=4)
    def _chunk(k0):
        for u in (0, 1, 2, 3):
            k = k0 + u
            b = u % 2
            j = u % 4
            j2 = (u + 2) % 4
            pltpu.make_async_copy(hs_hbm.at[idx[j].at[0]], rows[b], gsems[b]).wait()
            pltpu.sync_copy(rows[b], acc.at[idx[j].at[1]], add=True)

            @pl.when(k + 4 < NCH)
            def _refill():
                pltpu.async_copy(eidx_hbm.at[wid, k + 4], idx[j], isems[j])

            @pl.when(k + 2 < NCH)
            def _launch():
                pltpu.make_async_copy(eidx_hbm.at[wid, k + 2], idx[j2], isems[j2]).wait()
                pltpu.async_copy(hs_hbm.at[idx[j2].at[0]], rows[b], gsems[b])

    kl = NCH - 1
    bl = kl % 2
    jl = kl % 4
    pltpu.make_async_copy(hs_hbm.at[idx[jl].at[0]], rows[bl], gsems[bl]).wait()
    pltpu.sync_copy(rows[bl], acc.at[idx[jl].at[1]], add=True)

    plsc.subcore_barrier()

    @pl.when(c == 0)
    def _dump0():
        pltpu.sync_copy(acc.at[pl.ds(s * RPT, RPT)], p0_hbm.at[pl.ds(s * RPT, RPT)])

    @pl.when(c == 1)
    def _dump1():
        pltpu.sync_copy(acc.at[pl.ds(s * RPT, RPT)], p1_hbm.at[pl.ds(s * RPT, RPT)])


_spmm_kernel = pl.kernel(
    _spmm_body,
    out_type=[
        jax.ShapeDtypeStruct((NP, D), jnp.float32),
        jax.ShapeDtypeStruct((NP, D), jnp.float32),
    ],
    mesh=_sc_mesh,
    scratch_types=[
        pltpu.VMEM_SHARED((NP, D), jnp.float32),
        pltpu.VMEM((2, C), jnp.int32),
        pltpu.VMEM((2, C), jnp.int32),
        pltpu.VMEM((2, C), jnp.int32),
        pltpu.VMEM((2, C), jnp.int32),
        pltpu.VMEM((C, D), jnp.float32),
        pltpu.VMEM((C, D), jnp.float32),
        pltpu.SemaphoreType.DMA,
        pltpu.SemaphoreType.DMA,
        pltpu.SemaphoreType.DMA,
        pltpu.SemaphoreType.DMA,
        pltpu.SemaphoreType.DMA,
        pltpu.SemaphoreType.DMA,
    ],
)


def _prep_body(d0_ref, d1_ref, x_ref, dinv_ref, hs0_ref):
    deg = d0_ref[...] + d1_ref[...] + 1.0
    dinv = lax.rsqrt(deg)
    dinv_ref[...] = dinv
    hs0_ref[...] = dinv * x_ref[...]


_prep_kernel = pl.pallas_call(
    _prep_body,
    grid=(NB,),
    in_specs=[
        pl.BlockSpec((RB, 1), lambda i: (i, 0)),
        pl.BlockSpec((RB, 1), lambda i: (i, 0)),
        pl.BlockSpec((RB, D), lambda i: (i, 0)),
    ],
    out_specs=[
        pl.BlockSpec((RB, 1), lambda i: (i, 0)),
        pl.BlockSpec((RB, D), lambda i: (i, 0)),
    ],
    out_shape=[
        jax.ShapeDtypeStruct((NP, 1), jnp.float32),
        jax.ShapeDtypeStruct((NP, D), jnp.float32),
    ],
)


def _mid_body(dinv_ref, p0_ref, p1_ref, hs0_ref, h1_ref, hs1_ref):
    agg = p0_ref[...] + p1_ref[...] + hs0_ref[...]
    dinv = dinv_ref[...]
    h1 = dinv * agg
    h1_ref[...] = h1
    hs1_ref[...] = dinv * h1


_mid_kernel = pl.pallas_call(
    _mid_body,
    grid=(NB,),
    in_specs=[
        pl.BlockSpec((RB, 1), lambda i: (i, 0)),
        pl.BlockSpec((RB, D), lambda i: (i, 0)),
        pl.BlockSpec((RB, D), lambda i: (i, 0)),
        pl.BlockSpec((RB, D), lambda i: (i, 0)),
    ],
    out_specs=[
        pl.BlockSpec((RB, D), lambda i: (i, 0)),
        pl.BlockSpec((RB, D), lambda i: (i, 0)),
    ],
    out_shape=[
        jax.ShapeDtypeStruct((NP, D), jnp.float32),
        jax.ShapeDtypeStruct((NP, D), jnp.float32),
    ],
)


def _out_body(x_ref, h1_ref, q0_ref, q1_ref, hs1_ref, dinv_ref,
              w0_ref, w1_ref, w2_ref, b_ref, o_ref):
    h2 = dinv_ref[...] * (q0_ref[...] + q1_ref[...] + hs1_ref[...])
    acc = jnp.dot(x_ref[...], w0_ref[...], preferred_element_type=jnp.float32)
    acc = acc + jnp.dot(h1_ref[...], w1_ref[...], preferred_element_type=jnp.float32)
    acc = acc + jnp.dot(h2, w2_ref[...], preferred_element_type=jnp.float32)
    o_ref[...] = acc + b_ref[...]


_out_kernel = pl.pallas_call(
    _out_body,
    grid=(NB,),
    in_specs=[
        pl.BlockSpec((RB, D), lambda i: (i, 0)),
        pl.BlockSpec((RB, D), lambda i: (i, 0)),
        pl.BlockSpec((RB, D), lambda i: (i, 0)),
        pl.BlockSpec((RB, D), lambda i: (i, 0)),
        pl.BlockSpec((RB, D), lambda i: (i, 0)),
        pl.BlockSpec((RB, 1), lambda i: (i, 0)),
        pl.BlockSpec((D, D), lambda i: (0, 0)),
        pl.BlockSpec((D, D), lambda i: (0, 0)),
        pl.BlockSpec((D, D), lambda i: (0, 0)),
        pl.BlockSpec((1, D), lambda i: (0, 0)),
    ],
    out_specs=pl.BlockSpec((RB, D), lambda i: (i, 0)),
    out_shape=jax.ShapeDtypeStruct((NP, D), jnp.float32),
)


@jax.jit
def kernel(x, edge_index, W, b):
    col2 = edge_index[1].reshape(NW, NCH, C)
    eidx = edge_index.reshape(2, NW, NCH, C).transpose(1, 2, 0, 3)
    xp = jnp.pad(x, ((0, NP - N), (0, 0)))
    zrows = jnp.zeros((RPT, D), jnp.float32)

    d0, d1 = _deg_kernel(col2)
    dinv, hs0 = _prep_kernel(d0.reshape(NP, 1), d1.reshape(NP, 1), xp)
    p0, p1 = _spmm_kernel(hs0, eidx, zrows)
    h1, hs1 = _mid_kernel(dinv, p0, p1, hs0)
    q0, q1 = _spmm_kernel(hs1, eidx, zrows)
    Wt = W.T
    out = _out_kernel(xp, h1, q0, q1, hs1, dinv,
                      Wt[:D], Wt[D:2 * D], Wt[2 * D:], b.reshape(1, D))
    return out[:N]


# speedup vs baseline: 1.0821x; 1.0013x over previous
import functools

import jax
import jax.numpy as jnp
from jax import lax
from jax.experimental import pallas as pl
from jax.experimental.pallas import tpu as pltpu
from jax.experimental.pallas import tpu_sc as plsc

N = 10000
E = 320000
D = 128
NC = 2
NS = 16
NW = NC * NS
NP = 10240
RPT = NP // NS
EP = E // NW
C = 80
NCH = EP // C
NB = 16
RB = NP // NB

_sc_mesh = plsc.VectorSubcoreMesh(
    core_axis_name="c", subcore_axis_name="s", num_cores=NC, num_subcores=NS
)


def _deg_body(col2_hbm, deg0_hbm, deg1_hbm, acc, idx_a, ones_v, zero_v, isem):
    c = lax.axis_index("c")
    s = lax.axis_index("s")
    wid = c * NS + s

    pltpu.async_copy(col2_hbm.at[wid], idx_a, isem)

    @pl.loop(0, RPT // 16)
    def _zfill(i):
        zero_v[pl.ds(i * 16, 16)] = jnp.zeros((16,), jnp.float32)

    @pl.loop(0, C // 16)
    def _ofill(i):
        ones_v[pl.ds(i * 16, 16)] = jnp.ones((16,), jnp.float32)

    pltpu.sync_copy(zero_v, acc.at[pl.ds(s * RPT, RPT)])
    pltpu.make_async_copy(col2_hbm.at[wid], idx_a, isem).wait()
    plsc.subcore_barrier()

    @pl.loop(0, NCH)
    def _chunk(k):
        pltpu.sync_copy(ones_v, acc.at[idx_a.at[k]], add=True)

    plsc.subcore_barrier()

    @pl.when(c == 0)
    def _dump0():
        pltpu.sync_copy(acc.at[pl.ds(s * RPT, RPT)], deg0_hbm.at[pl.ds(s * RPT, RPT)])

    @pl.when(c == 1)
    def _dump1():
        pltpu.sync_copy(acc.at[pl.ds(s * RPT, RPT)], deg1_hbm.at[pl.ds(s * RPT, RPT)])


_deg_kernel = pl.kernel(
    _deg_body,
    out_type=[
        jax.ShapeDtypeStruct((NP,), jnp.float32),
        jax.ShapeDtypeStruct((NP,), jnp.float32),
    ],
    mesh=_sc_mesh,
    scratch_types=[
        pltpu.VMEM_SHARED((NP,), jnp.float32),
        pltpu.VMEM((NCH, C), jnp.int32),
        pltpu.VMEM((C,), jnp.float32),
        pltpu.VMEM((RPT,), jnp.float32),
        pltpu.SemaphoreType.DMA,
    ],
)


def _spmm_body(hs_hbm, eidx_hbm, z_hbm, p0_hbm, p1_hbm,
               acc, i0, i1, i2, i3, rows0, rows1,
               si0, si1, si2, si3, sg0, sg1):
    c = lax.axis_index("c")
    s = lax.axis_index("s")
    wid = c * NS + s

    idx = (i0, i1, i2, i3)
    isems = (si0, si1, si2, si3)
    rows = (rows0, rows1)
    gsems = (sg0, sg1)

    for j in (0, 1, 2, 3):
        pltpu.async_copy(eidx_hbm.at[wid, j], idx[j], isems[j])
    pltpu.sync_copy(z_hbm, acc.at[pl.ds(s * RPT, RPT)])
    for b in (0, 1):
        pltpu.make_async_copy(eidx_hbm.at[wid, b], idx[b], isems[b]).wait()
        pltpu.async_copy(hs_hbm.at[idx[b].at[0]], rows[b], gsems[b])
    plsc.subcore_barrier()

    @pl.loop(0, NCH - 1, step=4)
    def _chunk(k0):
        for u in (0, 1, 2, 3):
            k = k0 + u
            b = u % 2
            j = u % 4
            j2 = (u + 2) % 4
            pltpu.make_async_copy(hs_hbm.at[idx[j].at[0]], rows[b], gsems[b]).wait()
            pltpu.sync_copy(rows[b], acc.at[idx[j].at[1]], add=True)

            @pl.when(k + 4 < NCH)
            def _refill():
                pltpu.async_copy(eidx_hbm.at[wid, k + 4], idx[j], isems[j])

            @pl.when(k + 2 < NCH)
            def _launch():
                pltpu.make_async_copy(eidx_hbm.at[wid, k + 2], idx[j2], isems[j2]).wait()
                pltpu.async_copy(hs_hbm.at[idx[j2].at[0]], rows[b], gsems[b])

    kl = NCH - 1
    bl = kl % 2
    jl = kl % 4
    pltpu.make_async_copy(hs_hbm.at[idx[jl].at[0]], rows[bl], gsems[bl]).wait()
    pltpu.sync_copy(rows[bl], acc.at[idx[jl].at[1]], add=True)

    plsc.subcore_barrier()

    @pl.when(c == 0)
    def _dump0():
        pltpu.sync_copy(acc.at[pl.ds(s * RPT, RPT)], p0_hbm.at[pl.ds(s * RPT, RPT)])

    @pl.when(c == 1)
    def _dump1():
        pltpu.sync_copy(acc.at[pl.ds(s * RPT, RPT)], p1_hbm.at[pl.ds(s * RPT, RPT)])


_spmm_kernel = pl.kernel(
    _spmm_body,
    out_type=[
        jax.ShapeDtypeStruct((NP, D), jnp.float32),
        jax.ShapeDtypeStruct((NP, D), jnp.float32),
    ],
    mesh=_sc_mesh,
    scratch_types=[
        pltpu.VMEM_SHARED((NP, D), jnp.float32),
        pltpu.VMEM((2, C), jnp.int32),
        pltpu.VMEM((2, C), jnp.int32),
        pltpu.VMEM((2, C), jnp.int32),
        pltpu.VMEM((2, C), jnp.int32),
        pltpu.VMEM((C, D), jnp.float32),
        pltpu.VMEM((C, D), jnp.float32),
        pltpu.SemaphoreType.DMA,
        pltpu.SemaphoreType.DMA,
        pltpu.SemaphoreType.DMA,
        pltpu.SemaphoreType.DMA,
        pltpu.SemaphoreType.DMA,
        pltpu.SemaphoreType.DMA,
    ],
)


def _prep_body(d0_ref, d1_ref, x_ref, dinv_ref, hs0_ref):
    deg = d0_ref[...] + d1_ref[...] + 1.0
    dinv = lax.rsqrt(deg)
    dinv_ref[...] = dinv
    hs0_ref[...] = dinv * x_ref[...]


_prep_kernel = pl.pallas_call(
    _prep_body,
    grid=(NB,),
    in_specs=[
        pl.BlockSpec((RB, 1), lambda i: (i, 0)),
        pl.BlockSpec((RB, 1), lambda i: (i, 0)),
        pl.BlockSpec((RB, D), lambda i: (i, 0)),
    ],
    out_specs=[
        pl.BlockSpec((RB, 1), lambda i: (i, 0)),
        pl.BlockSpec((RB, D), lambda i: (i, 0)),
    ],
    out_shape=[
        jax.ShapeDtypeStruct((NP, 1), jnp.float32),
        jax.ShapeDtypeStruct((NP, D), jnp.float32),
    ],
)


def _mid_body(dinv_ref, p0_ref, p1_ref, hs0_ref, h1_ref, hs1_ref):
    agg = p0_ref[...] + p1_ref[...] + hs0_ref[...]
    dinv = dinv_ref[...]
    h1 = dinv * agg
    h1_ref[...] = h1
    hs1_ref[...] = dinv * h1


_mid_kernel = pl.pallas_call(
    _mid_body,
    grid=(NB,),
    in_specs=[
        pl.BlockSpec((RB, 1), lambda i: (i, 0)),
        pl.BlockSpec((RB, D), lambda i: (i, 0)),
        pl.BlockSpec((RB, D), lambda i: (i, 0)),
        pl.BlockSpec((RB, D), lambda i: (i, 0)),
    ],
    out_specs=[
        pl.BlockSpec((RB, D), lambda i: (i, 0)),
        pl.BlockSpec((RB, D), lambda i: (i, 0)),
    ],
    out_shape=[
        jax.ShapeDtypeStruct((NP, D), jnp.float32),
        jax.ShapeDtypeStruct((NP, D), jnp.float32),
    ],
)


def _out_body(x_ref, h1_ref, q0_ref, q1_ref, hs1_ref, dinv_ref,
              w0_ref, w1_ref, w2_ref, b_ref, o_ref):
    h2 = dinv_ref[...] * (q0_ref[...] + q1_ref[...] + hs1_ref[...])
    acc = jnp.dot(x_ref[...], w0_ref[...], preferred_element_type=jnp.float32)
    acc = acc + jnp.dot(h1_ref[...], w1_ref[...], preferred_element_type=jnp.float32)
    acc = acc + jnp.dot(h2, w2_ref[...], preferred_element_type=jnp.float32)
    o_ref[...] = acc + b_ref[...]


_out_kernel = pl.pallas_call(
    _out_body,
    grid=(NB,),
    in_specs=[
        pl.BlockSpec((RB, D), lambda i: (i, 0)),
        pl.BlockSpec((RB, D), lambda i: (i, 0)),
        pl.BlockSpec((RB, D), lambda i: (i, 0)),
        pl.BlockSpec((RB, D), lambda i: (i, 0)),
        pl.BlockSpec((RB, D), lambda i: (i, 0)),
        pl.BlockSpec((RB, 1), lambda i: (i, 0)),
        pl.BlockSpec((D, D), lambda i: (0, 0)),
        pl.BlockSpec((D, D), lambda i: (0, 0)),
        pl.BlockSpec((D, D), lambda i: (0, 0)),
        pl.BlockSpec((1, D), lambda i: (0, 0)),
    ],
    out_specs=pl.BlockSpec((RB, D), lambda i: (i, 0)),
    out_shape=jax.ShapeDtypeStruct((NP, D), jnp.float32),
)


@jax.jit
def kernel(x, edge_index, W, b):
    col2 = edge_index[1].reshape(NW, NCH, C)
    eidx = edge_index.reshape(2, NW, NCH, C).transpose(1, 2, 0, 3)
    xp = jnp.pad(x, ((0, NP - N), (0, 0)))
    zrows = jnp.zeros((RPT, D), jnp.float32)

    d0, d1 = _deg_kernel(col2)
    dinv, hs0 = _prep_kernel(d0.reshape(NP, 1), d1.reshape(NP, 1), xp)
    p0, p1 = _spmm_kernel(hs0, eidx, zrows)
    h1, hs1 = _mid_kernel(dinv, p0, p1, hs0)
    q0, q1 = _spmm_kernel(hs1, eidx, zrows)
    Wt = W.T
    out = _out_kernel(xp, h1, q0, q1, hs1, dinv,
                      Wt[:D], Wt[D:2 * D], Wt[2 * D:], b.reshape(1, D))
    return out[:N]
